# baseline (speedup 1.0000x reference)
import functools

import jax
import jax.numpy as jnp
import numpy as np
from jax import lax
from jax.experimental import pallas as pl
from jax.experimental.pallas import tpu as pltpu
from jax.experimental.pallas import tpu_sc as plsc

_RADIUS = 0.4
_R2 = _RADIUS * _RADIUS
_NS = 64
_BN_EPS = 1e-3

_NW = 32
_GRP = 8

_MB = 256
_NB = 2048
_RB = 4096


def _mesh():
    return plsc.VectorSubcoreMesh(core_axis_name="c", subcore_axis_name="s")


_SC_PARAMS = pltpu.CompilerParams(needs_layout_passes=False)


def _centers_call(xyzf, indsf, B, N, M):
    wpb = _NW // B
    mpw = M // wpb
    npw = N // wpb

    @functools.partial(
        pl.kernel,
        out_type=(jax.ShapeDtypeStruct((B * M * 3,), jnp.float32),
                  jax.ShapeDtypeStruct((B * 3 * N,), jnp.float32)),
        mesh=_mesh(),
        compiler_params=_SC_PARAMS,
        scratch_types=[
            pltpu.VMEM((3 * N,), jnp.float32),
            pltpu.VMEM((mpw,), jnp.int32),
            pltpu.VMEM((mpw * 3,), jnp.float32),
            pltpu.VMEM((npw,), jnp.float32),
        ],
    )
    def body(xyz_hbm, inds_hbm, nxyz_hbm, xyzt_hbm, pb, ib, nb, tb):
        w = lax.axis_index("s") * 2 + lax.axis_index("c")
        b = w // wpb
        k = w % wpb
        pltpu.sync_copy(xyz_hbm.at[pl.ds(b * 3 * N, 3 * N)], pb)
        pltpu.sync_copy(inds_hbm.at[pl.ds(b * M + k * mpw, mpw)], ib)
        iota = lax.iota(jnp.int32, 16)
        for j in range(mpw // 16):
            civ = ib[pl.ds(j * 16, 16)] * 3
            base3 = (iota + j * 16) * 3
            plsc.store_scatter(nb, [base3], plsc.load_gather(pb, [civ]))
            plsc.store_scatter(nb, [base3 + 1], plsc.load_gather(pb, [civ + 1]))
            plsc.store_scatter(nb, [base3 + 2], plsc.load_gather(pb, [civ + 2]))
        pltpu.sync_copy(nb, nxyz_hbm.at[pl.ds((b * M + k * mpw) * 3, mpw * 3)])
        for c in range(3):
            for j in range(npw // 16):
                src = (k * npw + j * 16 + iota) * 3 + c
                tb[pl.ds(j * 16, 16)] = plsc.load_gather(pb, [src])
            pltpu.sync_copy(tb, xyzt_hbm.at[pl.ds((b * 3 + c) * N + k * npw, npw)])

    return body(xyzf, indsf)


def _sq_body(nx_ref, xt_ref, p_ref, out_ref):
    a = nx_ref[0]
    bt = xt_ref[0]
    dot = lax.dot_general(
        a.astype(jnp.bfloat16), bt.astype(jnp.bfloat16),
        (((1,), (0,)), ((), ())), preferred_element_type=jnp.float32)
    cn = jnp.sum(a * a, axis=1)[:, None]
    pn = jnp.sum(bt * bt, axis=0)[None, :]
    sq = cn + pn - 2.0 * dot
    mb = (sq < _R2).astype(jnp.bfloat16)
    packed = lax.dot_general(
        mb, p_ref[...],
        (((1,), (0,)), ((), ())), preferred_element_type=jnp.float32)
    out_ref[0] = packed.astype(jnp.int32)


def _mask_pallas(new_xyz, xyz_t, pmat):
    B, M, _ = new_xyz.shape
    N = xyz_t.shape[2]
    nw = N // 16
    return pl.pallas_call(
        _sq_body,
        grid=(B, M // _MB, N // _NB),
        in_specs=[
            pl.BlockSpec((1, _MB, 3), lambda b, i, j: (b, i, 0)),
            pl.BlockSpec((1, 3, _NB), lambda b, i, j: (b, 0, j)),
            pl.BlockSpec((_NB, _NB // 16), lambda b, i, j: (0, 0)),
        ],
        out_specs=pl.BlockSpec((1, _MB, _NB // 16), lambda b, i, j: (b, i, j)),
        out_shape=jax.ShapeDtypeStruct((B, M, nw), jnp.int32),
    )(new_xyz, xyz_t, pmat)


def _group_call(maskf, xyzf, featf, indsf, B, N, M):
    wpb = _NW // B
    mpw = M // wpb
    nw = N // 16
    nwv = nw // 16

    @functools.partial(
        pl.kernel,
        out_type=(jax.ShapeDtypeStruct((B * M * _NS * 4,), jnp.float32),
                  jax.ShapeDtypeStruct((4 * B * M * _NS,), jnp.float32)),
        mesh=_mesh(),
        compiler_params=_SC_PARAMS,
        scratch_types=[
            pltpu.VMEM((3 * N,), jnp.float32),
            pltpu.VMEM((N,), jnp.float32),
            pltpu.VMEM((mpw,), jnp.int32),
            pltpu.VMEM((mpw + 16,), jnp.float32),
            pltpu.VMEM((mpw + 16,), jnp.float32),
            pltpu.VMEM((mpw + 16,), jnp.float32),
            pltpu.VMEM((_GRP * nw,), jnp.int32),
            pltpu.VMEM((_NS + 16,), jnp.int32),
            pltpu.VMEM((mpw * _NS * 2,), jnp.float32),
            pltpu.VMEM((2 * mpw * _NS,), jnp.float32),
        ],
    )
    def body(mask_hbm, xyz_hbm, feat_hbm, inds_hbm, out_hbm, xpl_hbm,
             pb, fs, ib, cxb, cyb, czb, mkb, idxb, gb, px):
        w = lax.axis_index("s") * 2 + lax.axis_index("c")
        b = w // wpb
        k = w % wpb
        pltpu.sync_copy(xyz_hbm.at[pl.ds(b * 3 * N, 3 * N)], pb)
        pltpu.sync_copy(feat_hbm.at[pl.ds(b * N, N)], fs)
        pltpu.sync_copy(inds_hbm.at[pl.ds(b * M + k * mpw, mpw)], ib)
        iota = lax.iota(jnp.int32, 16)
        for j in range(mpw // 16):
            civ = ib[pl.ds(j * 16, 16)] * 3
            cxb[pl.ds(j * 16, 16)] = plsc.load_gather(pb, [civ])
            cyb[pl.ds(j * 16, 16)] = plsc.load_gather(pb, [civ + 1])
            czb[pl.ds(j * 16, 16)] = plsc.load_gather(pb, [civ + 2])
        row0 = b * M + k * mpw

        def make_group_body(h):
          def group_body(g, carry0):
            pltpu.sync_copy(
                mask_hbm.at[pl.ds((row0 + h * (mpw // 2) + g * _GRP) * nw,
                                  _GRP * nw)], mkb)

            def center_body(ci8, carry):
                ci = h * (mpw // 2) + g * _GRP + ci8
                cl = g * _GRP + ci8
                cx = cxb[pl.ds(ci, 16)][0]
                cy = cyb[pl.ds(ci, 16)][0]
                cz = czb[pl.ds(ci, 16)][0]

                def cond(st):
                    wv_i, cnt = st
                    return (cnt < _NS) & (wv_i < nwv)

                def wbody(st):
                    wv_i, cnt = st
                    wv = mkb[pl.ds(ci8 * nw + wv_i * 16, 16)]
                    base = wv_i * 256
                    for j in range(16):
                        m = ((wv[j] >> iota) & 1) == 1
                        pidx = base + j * 16 + iota
                        plsc.store_compressed(
                            idxb.at[pl.ds(jnp.minimum(cnt, _NS), 16)],
                            pidx, mask=m)
                        cnt = cnt + plsc.all_reduce_population_count(m)[0]
                    return wv_i + 1, cnt

                _, cnt = lax.while_loop(cond, wbody,
                                        (jnp.int32(0), jnp.int32(0)))
                first = idxb[pl.ds(0, 16)][0]
                gbase = cl * (_NS * 4)
                for t in range(_NS // 16):
                    pos = iota + t * 16
                    iv = idxb[pl.ds(t * 16, 16)]
                    iv = jnp.where(pos < cnt, iv, first)
                    iv3 = iv * 3
                    gx = (plsc.load_gather(pb, [iv3]) - cx) / _RADIUS
                    gy = (plsc.load_gather(pb, [iv3 + 1]) - cy) / _RADIUS
                    gz = (plsc.load_gather(pb, [iv3 + 2]) - cz) / _RADIUS
                    gf = plsc.load_gather(fs, [iv])
                    p4 = gbase + pos * 4
                    plsc.store_scatter(gb, [p4], gx)
                    plsc.store_scatter(gb, [p4 + 1], gy)
                    plsc.store_scatter(gb, [p4 + 2], gz)
                    plsc.store_scatter(gb, [p4 + 3], gf)
                    hns = (mpw // 2) * _NS
                    rbase = cl * _NS + t * 16
                    px[pl.ds(0 * hns + rbase, 16)] = gx
                    px[pl.ds(1 * hns + rbase, 16)] = gy
                    px[pl.ds(2 * hns + rbase, 16)] = gz
                    px[pl.ds(3 * hns + rbase, 16)] = gf
                return carry

            return lax.fori_loop(0, _GRP, center_body, carry0)
          return group_body

        hns = (mpw // 2) * _NS
        for h in range(2):
            lax.fori_loop(0, mpw // 2 // _GRP, make_group_body(h), jnp.int32(0))
            hrow = row0 + h * (mpw // 2)
            pltpu.sync_copy(gb, out_hbm.at[pl.ds(hrow * (_NS * 4), hns * 4)])
            for c in range(4):
                pltpu.sync_copy(
                    px.at[pl.ds(c * hns, hns)],
                    xpl_hbm.at[pl.ds(c * (B * M * _NS) + hrow * _NS, hns)])

    return body(maskf, xyzf, featf, indsf)


def _mlp_body(x_ref, w0_ref, b0_ref, g0_ref, be0_ref, w1_ref, b1_ref, g1_ref,
              be1_ref, w2_ref, b2_ref, g2_ref, be2_ref, out_ref):
    sq_bn = jnp.sqrt(jnp.float32(1.0 + _BN_EPS))
    h = jnp.transpose(x_ref[...], (1, 0))
    for wr, br, gr, ber in ((w0_ref, b0_ref, g0_ref, be0_ref),
                            (w1_ref, b1_ref, g1_ref, be1_ref),
                            (w2_ref, b2_ref, g2_ref, be2_ref)):
        y = lax.dot_general(
            h.astype(jnp.bfloat16), wr[...].astype(jnp.bfloat16),
            (((1,), (0,)), ((), ())), preferred_element_type=jnp.float32)
        y = y + br[...]
        y = gr[...] * (y / sq_bn) + ber[...]
        h = jnp.clip(y, 0.0, 6.0)
    hm = h.reshape(_RB // _NS, _NS, h.shape[-1])
    out_ref[...] = jnp.max(hm, axis=1)


def _mlp_call(x, params):
    rows = x.shape[1]
    full = lambda shape: pl.BlockSpec(shape, lambda i: (0,) * len(shape))
    in_specs = [pl.BlockSpec((4, _RB), lambda i: (0, i))]
    args = [x]
    for (w, b, g, be) in params:
        in_specs += [full(w.shape), full(b.shape), full(g.shape), full(be.shape)]
        args += [w, b, g, be]
    return pl.pallas_call(
        _mlp_body,
        grid=(rows // _RB,),
        in_specs=in_specs,
        out_specs=pl.BlockSpec((_RB // _NS, 128), lambda i: (i, 0)),
        out_shape=jax.ShapeDtypeStruct((rows // _NS, 128), jnp.float32),
    )(*args)


def _bit_weights():
    p = np.zeros((_NB, _NB // 16), np.float32)
    n = np.arange(_NB)
    p[n, n // 16] = 2.0 ** (n % 16)
    return jnp.asarray(p, dtype=jnp.bfloat16)


def kernel(xyz, features, inds, W0, b0, g0, be0, W1, b1, g1, be1, W2, b2, g2, be2):
    B, N, _ = xyz.shape
    M = inds.shape[1]
    xyzf = xyz.reshape(-1)
    featf = features.reshape(-1)
    indsf = inds.reshape(-1)

    newxyz_flat, xyzt_flat = _centers_call(xyzf, indsf, B, N, M)
    new_xyz = newxyz_flat.reshape(B, M, 3)
    xyz_t = xyzt_flat.reshape(B, 3, N)

    maskw = _mask_pallas(new_xyz, xyz_t, _bit_weights())

    grouped_flat, xplanes = _group_call(maskw.reshape(-1), xyzf, featf, indsf,
                                        B, N, M)
    grouped_features = grouped_flat.reshape(B, M, _NS, 4)

    params = [(W0, b0.reshape(1, -1), g0.reshape(1, -1), be0.reshape(1, -1)),
              (W1, b1.reshape(1, -1), g1.reshape(1, -1), be1.reshape(1, -1)),
              (W2, b2.reshape(1, -1), g2.reshape(1, -1), be2.reshape(1, -1))]
    nf = _mlp_call(xplanes.reshape(4, B * M * _NS), params)
    new_features = nf.reshape(B, M, 128)

    return (new_xyz, new_features, inds, grouped_features)


# speedup vs baseline: 1.3055x; 1.3055x over previous
import functools

import jax
import jax.numpy as jnp
import numpy as np
from jax import lax
from jax.experimental import pallas as pl
from jax.experimental.pallas import tpu as pltpu
from jax.experimental.pallas import tpu_sc as plsc

_RADIUS = 0.4
_R2 = _RADIUS * _RADIUS
_NS = 64
_BN_EPS = 1e-3

_NW = 32
_GRP = 8

_MB = 256
_NB = 2048
_RB = 4096


def _mesh():
    return plsc.VectorSubcoreMesh(core_axis_name="c", subcore_axis_name="s")


_SC_PARAMS = pltpu.CompilerParams(needs_layout_passes=False)


def _centers_call(xyzf, indsf, B, N, M):
    wpb = _NW // B
    mpw = M // wpb
    npw = N // wpb

    @functools.partial(
        pl.kernel,
        out_type=(jax.ShapeDtypeStruct((B * M * 3,), jnp.float32),
                  jax.ShapeDtypeStruct((B * 3 * N,), jnp.float32)),
        mesh=_mesh(),
        compiler_params=_SC_PARAMS,
        scratch_types=[
            pltpu.VMEM((3 * N,), jnp.float32),
            pltpu.VMEM((mpw,), jnp.int32),
            pltpu.VMEM((mpw * 3,), jnp.float32),
            pltpu.VMEM((npw,), jnp.float32),
        ],
    )
    def body(xyz_hbm, inds_hbm, nxyz_hbm, xyzt_hbm, pb, ib, nb, tb):
        w = lax.axis_index("s") * 2 + lax.axis_index("c")
        b = w // wpb
        k = w % wpb
        pltpu.sync_copy(xyz_hbm.at[pl.ds(b * 3 * N, 3 * N)], pb)
        pltpu.sync_copy(inds_hbm.at[pl.ds(b * M + k * mpw, mpw)], ib)
        iota = lax.iota(jnp.int32, 16)
        for j in range(mpw // 16):
            civ = ib[pl.ds(j * 16, 16)] * 3
            base3 = (iota + j * 16) * 3
            plsc.store_scatter(nb, [base3], plsc.load_gather(pb, [civ]))
            plsc.store_scatter(nb, [base3 + 1], plsc.load_gather(pb, [civ + 1]))
            plsc.store_scatter(nb, [base3 + 2], plsc.load_gather(pb, [civ + 2]))
        pltpu.sync_copy(nb, nxyz_hbm.at[pl.ds((b * M + k * mpw) * 3, mpw * 3)])
        for c in range(3):
            for j in range(npw // 16):
                src = (k * npw + j * 16 + iota) * 3 + c
                tb[pl.ds(j * 16, 16)] = plsc.load_gather(pb, [src])
            pltpu.sync_copy(tb, xyzt_hbm.at[pl.ds((b * 3 + c) * N + k * npw, npw)])

    return body(xyzf, indsf)


def _sq_body(nx_ref, xt_ref, p_ref, out_ref):
    a = nx_ref[0]
    bt = xt_ref[0]
    dot = lax.dot_general(
        a.astype(jnp.bfloat16), bt.astype(jnp.bfloat16),
        (((1,), (0,)), ((), ())), preferred_element_type=jnp.float32)
    cn = jnp.sum(a * a, axis=1)[:, None]
    pn = jnp.sum(bt * bt, axis=0)[None, :]
    sq = cn + pn - 2.0 * dot
    mb = (sq < _R2).astype(jnp.bfloat16)
    packed = lax.dot_general(
        mb, p_ref[...],
        (((1,), (0,)), ((), ())), preferred_element_type=jnp.float32)
    out_ref[0] = packed.astype(jnp.int32)


def _mask_pallas(new_xyz, xyz_t, pmat):
    B, M, _ = new_xyz.shape
    N = xyz_t.shape[2]
    nw = N // 16
    return pl.pallas_call(
        _sq_body,
        grid=(B, M // _MB, N // _NB),
        in_specs=[
            pl.BlockSpec((1, _MB, 3), lambda b, i, j: (b, i, 0)),
            pl.BlockSpec((1, 3, _NB), lambda b, i, j: (b, 0, j)),
            pl.BlockSpec((_NB, _NB // 16), lambda b, i, j: (0, 0)),
        ],
        out_specs=pl.BlockSpec((1, _MB, _NB // 16), lambda b, i, j: (b, i, j)),
        out_shape=jax.ShapeDtypeStruct((B, M, nw), jnp.int32),
    )(new_xyz, xyz_t, pmat)


def _group_call(maskf, xyzf, featf, indsf, B, N, M):
    wpb = _NW // B
    mpw = M // wpb
    nw = N // 16
    nwv = nw // 16

    @functools.partial(
        pl.kernel,
        out_type=jax.ShapeDtypeStruct((4 * B * M * _NS,), jnp.float32),
        mesh=_mesh(),
        compiler_params=_SC_PARAMS,
        scratch_types=[
            pltpu.VMEM((3 * N,), jnp.float32),
            pltpu.VMEM((N,), jnp.float32),
            pltpu.VMEM((mpw,), jnp.int32),
            pltpu.VMEM((mpw + 16,), jnp.float32),
            pltpu.VMEM((mpw + 16,), jnp.float32),
            pltpu.VMEM((mpw + 16,), jnp.float32),
            pltpu.VMEM((_GRP * nw,), jnp.int32),
            pltpu.VMEM((_NS + 16,), jnp.int32),
            pltpu.VMEM((2 * mpw * _NS,), jnp.float32),
        ],
    )
    def body(mask_hbm, xyz_hbm, feat_hbm, inds_hbm, xpl_hbm,
             pb, fs, ib, cxb, cyb, czb, mkb, idxb, px):
        w = lax.axis_index("s") * 2 + lax.axis_index("c")
        b = w // wpb
        k = w % wpb
        pltpu.sync_copy(xyz_hbm.at[pl.ds(b * 3 * N, 3 * N)], pb)
        pltpu.sync_copy(feat_hbm.at[pl.ds(b * N, N)], fs)
        pltpu.sync_copy(inds_hbm.at[pl.ds(b * M + k * mpw, mpw)], ib)
        iota = lax.iota(jnp.int32, 16)
        for j in range(mpw // 16):
            civ = ib[pl.ds(j * 16, 16)] * 3
            cxb[pl.ds(j * 16, 16)] = plsc.load_gather(pb, [civ])
            cyb[pl.ds(j * 16, 16)] = plsc.load_gather(pb, [civ + 1])
            czb[pl.ds(j * 16, 16)] = plsc.load_gather(pb, [civ + 2])
        row0 = b * M + k * mpw

        def make_group_body(h):
          def group_body(g, carry0):
            pltpu.sync_copy(
                mask_hbm.at[pl.ds((row0 + h * (mpw // 2) + g * _GRP) * nw,
                                  _GRP * nw)], mkb)

            def center_body(ci8, carry):
                ci = h * (mpw // 2) + g * _GRP + ci8
                cl = g * _GRP + ci8
                cx = cxb[pl.ds(ci, 16)][0]
                cy = cyb[pl.ds(ci, 16)][0]
                cz = czb[pl.ds(ci, 16)][0]

                def cond(st):
                    wv_i, cnt = st
                    return (cnt < _NS) & (wv_i < nwv)

                def wbody(st):
                    wv_i, cnt = st
                    wv = mkb[pl.ds(ci8 * nw + wv_i * 16, 16)]
                    base = wv_i * 256
                    for j in range(16):
                        m = ((wv[j] >> iota) & 1) == 1
                        pidx = base + j * 16 + iota
                        plsc.store_compressed(
                            idxb.at[pl.ds(jnp.minimum(cnt, _NS), 16)],
                            pidx, mask=m)
                        cnt = cnt + plsc.all_reduce_population_count(m)[0]
                    return wv_i + 1, cnt

                _, cnt = lax.while_loop(cond, wbody,
                                        (jnp.int32(0), jnp.int32(0)))
                first = idxb[pl.ds(0, 16)][0]
                for t in range(_NS // 16):
                    pos = iota + t * 16
                    iv = idxb[pl.ds(t * 16, 16)]
                    iv = jnp.where(pos < cnt, iv, first)
                    iv3 = iv * 3
                    gx = (plsc.load_gather(pb, [iv3]) - cx) / _RADIUS
                    gy = (plsc.load_gather(pb, [iv3 + 1]) - cy) / _RADIUS
                    gz = (plsc.load_gather(pb, [iv3 + 2]) - cz) / _RADIUS
                    gf = plsc.load_gather(fs, [iv])
                    hns = (mpw // 2) * _NS
                    rbase = cl * _NS + t * 16
                    px[pl.ds(0 * hns + rbase, 16)] = gx
                    px[pl.ds(1 * hns + rbase, 16)] = gy
                    px[pl.ds(2 * hns + rbase, 16)] = gz
                    px[pl.ds(3 * hns + rbase, 16)] = gf
                return carry

            return lax.fori_loop(0, _GRP, center_body, carry0)
          return group_body

        hns = (mpw // 2) * _NS
        for h in range(2):
            lax.fori_loop(0, mpw // 2 // _GRP, make_group_body(h), jnp.int32(0))
            hrow = row0 + h * (mpw // 2)
            for c in range(4):
                pltpu.sync_copy(
                    px.at[pl.ds(c * hns, hns)],
                    xpl_hbm.at[pl.ds(c * (B * M * _NS) + hrow * _NS, hns)])

    return body(maskf, xyzf, featf, indsf)


def _mlp_body(x_ref, w0_ref, b0_ref, g0_ref, be0_ref, w1_ref, b1_ref, g1_ref,
              be1_ref, w2_ref, b2_ref, g2_ref, be2_ref, out_ref, grp_ref):
    sq_bn = jnp.sqrt(jnp.float32(1.0 + _BN_EPS))
    h = jnp.transpose(x_ref[...], (1, 0))
    grp_ref[...] = h.reshape(_RB // _NS, _NS, 4)
    for wr, br, gr, ber in ((w0_ref, b0_ref, g0_ref, be0_ref),
                            (w1_ref, b1_ref, g1_ref, be1_ref),
                            (w2_ref, b2_ref, g2_ref, be2_ref)):
        y = lax.dot_general(
            h.astype(jnp.bfloat16), wr[...].astype(jnp.bfloat16),
            (((1,), (0,)), ((), ())), preferred_element_type=jnp.float32)
        y = y + br[...]
        y = gr[...] * (y / sq_bn) + ber[...]
        h = jnp.clip(y, 0.0, 6.0)
    hm = h.reshape(_RB // _NS, _NS, h.shape[-1])
    out_ref[...] = jnp.max(hm, axis=1)


def _mlp_call(x, params):
    rows = x.shape[1]
    full = lambda shape: pl.BlockSpec(shape, lambda i: (0,) * len(shape))
    in_specs = [pl.BlockSpec((4, _RB), lambda i: (0, i))]
    args = [x]
    for (w, b, g, be) in params:
        in_specs += [full(w.shape), full(b.shape), full(g.shape), full(be.shape)]
        args += [w, b, g, be]
    return pl.pallas_call(
        _mlp_body,
        grid=(rows // _RB,),
        in_specs=in_specs,
        out_specs=[pl.BlockSpec((_RB // _NS, 128), lambda i: (i, 0)),
                   pl.BlockSpec((_RB // _NS, _NS, 4), lambda i: (i, 0, 0))],
        out_shape=[jax.ShapeDtypeStruct((rows // _NS, 128), jnp.float32),
                   jax.ShapeDtypeStruct((rows // _NS, _NS, 4), jnp.float32)],
    )(*args)


def _bit_weights():
    p = np.zeros((_NB, _NB // 16), np.float32)
    n = np.arange(_NB)
    p[n, n // 16] = 2.0 ** (n % 16)
    return jnp.asarray(p, dtype=jnp.bfloat16)


def kernel(xyz, features, inds, W0, b0, g0, be0, W1, b1, g1, be1, W2, b2, g2, be2):
    B, N, _ = xyz.shape
    M = inds.shape[1]
    xyzf = xyz.reshape(-1)
    featf = features.reshape(-1)
    indsf = inds.reshape(-1)

    newxyz_flat, xyzt_flat = _centers_call(xyzf, indsf, B, N, M)
    new_xyz = newxyz_flat.reshape(B, M, 3)
    xyz_t = xyzt_flat.reshape(B, 3, N)

    maskw = _mask_pallas(new_xyz, xyz_t, _bit_weights())

    xplanes = _group_call(maskw.reshape(-1), xyzf, featf, indsf, B, N, M)

    params = [(W0, b0.reshape(1, -1), g0.reshape(1, -1), be0.reshape(1, -1)),
              (W1, b1.reshape(1, -1), g1.reshape(1, -1), be1.reshape(1, -1)),
              (W2, b2.reshape(1, -1), g2.reshape(1, -1), be2.reshape(1, -1))]
    nf, grouped = _mlp_call(xplanes.reshape(4, B * M * _NS), params)
    new_features = nf.reshape(B, M, 128)
    grouped_features = grouped.reshape(B, M, _NS, 4)

    return (new_xyz, new_features, inds, grouped_features)


# speedup vs baseline: 1.3088x; 1.0025x over previous
import functools

import jax
import jax.numpy as jnp
import numpy as np
from jax import lax
from jax.experimental import pallas as pl
from jax.experimental.pallas import tpu as pltpu
from jax.experimental.pallas import tpu_sc as plsc

_RADIUS = 0.4
_R2 = _RADIUS * _RADIUS
_NS = 64
_BN_EPS = 1e-3

_NW = 32
_GRP = 8

_MB = 256
_NB = 2048
_RB = 4096


def _mesh():
    return plsc.VectorSubcoreMesh(core_axis_name="c", subcore_axis_name="s")


_SC_PARAMS = pltpu.CompilerParams(needs_layout_passes=False)


def _centers_call(xyzf, indsf, B, N, M):
    wpb = _NW // B
    mpw = M // wpb
    npw = N // wpb

    @functools.partial(
        pl.kernel,
        out_type=(jax.ShapeDtypeStruct((3 * B * M,), jnp.float32),
                  jax.ShapeDtypeStruct((B * 3 * N,), jnp.float32)),
        mesh=_mesh(),
        compiler_params=_SC_PARAMS,
        scratch_types=[
            pltpu.VMEM((3 * N,), jnp.float32),
            pltpu.VMEM((mpw,), jnp.int32),
            pltpu.VMEM((mpw * 3,), jnp.float32),
            pltpu.VMEM((npw,), jnp.float32),
        ],
    )
    def body(xyz_hbm, inds_hbm, nxyz_hbm, xyzt_hbm, pb, ib, nb, tb):
        w = lax.axis_index("s") * 2 + lax.axis_index("c")
        b = w // wpb
        k = w % wpb
        pltpu.sync_copy(xyz_hbm.at[pl.ds(b * 3 * N, 3 * N)], pb)
        pltpu.sync_copy(inds_hbm.at[pl.ds(b * M + k * mpw, mpw)], ib)
        iota = lax.iota(jnp.int32, 16)
        for j in range(mpw // 16):
            civ = ib[pl.ds(j * 16, 16)] * 3
            for c in range(3):
                nb[pl.ds(c * mpw + j * 16, 16)] = plsc.load_gather(pb, [civ + c])
        for c in range(3):
            pltpu.sync_copy(nb.at[pl.ds(c * mpw, mpw)],
                            nxyz_hbm.at[pl.ds((b * 3 + c) * M + k * mpw, mpw)])
        for c in range(3):
            for j in range(npw // 16):
                src = (k * npw + j * 16 + iota) * 3 + c
                tb[pl.ds(j * 16, 16)] = plsc.load_gather(pb, [src])
            pltpu.sync_copy(tb, xyzt_hbm.at[pl.ds((b * 3 + c) * N + k * npw, npw)])

    return body(xyzf, indsf)


def _sq_body(nx_ref, xt_ref, p_ref, out_ref, nxyz_ref):
    ap = nx_ref[0]
    a = jnp.transpose(ap, (1, 0))
    bt = xt_ref[0]
    nxyz_ref[0] = a
    a2 = a.astype(jnp.bfloat16) * jnp.bfloat16(2.0)
    dot2 = lax.dot_general(
        a2, bt.astype(jnp.bfloat16),
        (((1,), (0,)), ((), ())), preferred_element_type=jnp.float32)
    cn = jnp.sum(a * a, axis=1)[:, None]
    pn = jnp.sum(bt * bt, axis=0)[None, :]
    sq = cn + pn - dot2
    mb = (sq < _R2).astype(jnp.bfloat16)
    packed = lax.dot_general(
        mb, p_ref[...],
        (((1,), (0,)), ((), ())), preferred_element_type=jnp.float32)
    out_ref[0] = packed.astype(jnp.int32)


def _mask_pallas(nxyz_planes, xyz_t, pmat):
    B, _, M = nxyz_planes.shape
    N = xyz_t.shape[2]
    nw = N // 16
    return pl.pallas_call(
        _sq_body,
        grid=(B, M // _MB, N // _NB),
        in_specs=[
            pl.BlockSpec((1, 3, _MB), lambda b, i, j: (b, 0, i)),
            pl.BlockSpec((1, 3, _NB), lambda b, i, j: (b, 0, j)),
            pl.BlockSpec((_NB, _NB // 16), lambda b, i, j: (0, 0)),
        ],
        out_specs=[pl.BlockSpec((1, _MB, _NB // 16), lambda b, i, j: (b, i, j)),
                   pl.BlockSpec((1, _MB, 3), lambda b, i, j: (b, i, 0))],
        out_shape=[jax.ShapeDtypeStruct((B, M, nw), jnp.int32),
                   jax.ShapeDtypeStruct((B, M, 3), jnp.float32)],
    )(nxyz_planes, xyz_t, pmat)


def _group_call(maskf, xyzf, featf, indsf, B, N, M):
    wpb = _NW // B
    mpw = M // wpb
    nw = N // 16
    nwv = nw // 16

    @functools.partial(
        pl.kernel,
        out_type=jax.ShapeDtypeStruct((4 * B * M * _NS,), jnp.float32),
        mesh=_mesh(),
        compiler_params=_SC_PARAMS,
        scratch_types=[
            pltpu.VMEM((3 * N,), jnp.float32),
            pltpu.VMEM((N,), jnp.float32),
            pltpu.VMEM((mpw,), jnp.int32),
            pltpu.VMEM((mpw + 16,), jnp.float32),
            pltpu.VMEM((mpw + 16,), jnp.float32),
            pltpu.VMEM((mpw + 16,), jnp.float32),
            pltpu.VMEM((_GRP * nw,), jnp.int32),
            pltpu.VMEM((_NS + 16,), jnp.int32),
            pltpu.VMEM((2 * mpw * _NS,), jnp.float32),
        ],
    )
    def body(mask_hbm, xyz_hbm, feat_hbm, inds_hbm, xpl_hbm,
             pb, fs, ib, cxb, cyb, czb, mkb, idxb, px):
        w = lax.axis_index("s") * 2 + lax.axis_index("c")
        b = w // wpb
        k = w % wpb
        pltpu.sync_copy(xyz_hbm.at[pl.ds(b * 3 * N, 3 * N)], pb)
        pltpu.sync_copy(feat_hbm.at[pl.ds(b * N, N)], fs)
        pltpu.sync_copy(inds_hbm.at[pl.ds(b * M + k * mpw, mpw)], ib)
        iota = lax.iota(jnp.int32, 16)
        for j in range(mpw // 16):
            civ = ib[pl.ds(j * 16, 16)] * 3
            cxb[pl.ds(j * 16, 16)] = plsc.load_gather(pb, [civ])
            cyb[pl.ds(j * 16, 16)] = plsc.load_gather(pb, [civ + 1])
            czb[pl.ds(j * 16, 16)] = plsc.load_gather(pb, [civ + 2])
        row0 = b * M + k * mpw

        def make_group_body(h):
          def group_body(g, carry0):
            pltpu.sync_copy(
                mask_hbm.at[pl.ds((row0 + h * (mpw // 2) + g * _GRP) * nw,
                                  _GRP * nw)], mkb)

            def center_body(ci8, carry):
                ci = h * (mpw // 2) + g * _GRP + ci8
                cl = g * _GRP + ci8
                cx = cxb[pl.ds(ci, 16)][0]
                cy = cyb[pl.ds(ci, 16)][0]
                cz = czb[pl.ds(ci, 16)][0]

                def cond(st):
                    wv_i, cnt = st
                    return (cnt < _NS) & (wv_i < nwv)

                def wbody(st):
                    wv_i, cnt = st
                    wv = mkb[pl.ds(ci8 * nw + wv_i * 16, 16)]
                    base = wv_i * 256
                    for j in range(16):
                        m = ((wv[j] >> iota) & 1) == 1
                        pidx = base + j * 16 + iota
                        plsc.store_compressed(
                            idxb.at[pl.ds(jnp.minimum(cnt, _NS), 16)],
                            pidx, mask=m)
                        cnt = cnt + plsc.all_reduce_population_count(m)[0]
                    return wv_i + 1, cnt

                _, cnt = lax.while_loop(cond, wbody,
                                        (jnp.int32(0), jnp.int32(0)))
                first = idxb[pl.ds(0, 16)][0]
                for t in range(_NS // 16):
                    pos = iota + t * 16
                    iv = idxb[pl.ds(t * 16, 16)]
                    iv = jnp.where(pos < cnt, iv, first)
                    iv3 = iv * 3
                    gx = (plsc.load_gather(pb, [iv3]) - cx) / _RADIUS
                    gy = (plsc.load_gather(pb, [iv3 + 1]) - cy) / _RADIUS
                    gz = (plsc.load_gather(pb, [iv3 + 2]) - cz) / _RADIUS
                    gf = plsc.load_gather(fs, [iv])
                    hns = (mpw // 2) * _NS
                    rbase = cl * _NS + t * 16
                    px[pl.ds(0 * hns + rbase, 16)] = gx
                    px[pl.ds(1 * hns + rbase, 16)] = gy
                    px[pl.ds(2 * hns + rbase, 16)] = gz
                    px[pl.ds(3 * hns + rbase, 16)] = gf
                return carry

            return lax.fori_loop(0, _GRP, center_body, carry0)
          return group_body

        hns = (mpw // 2) * _NS
        for h in range(2):
            lax.fori_loop(0, mpw // 2 // _GRP, make_group_body(h), jnp.int32(0))
            hrow = row0 + h * (mpw // 2)
            for c in range(4):
                pltpu.sync_copy(
                    px.at[pl.ds(c * hns, hns)],
                    xpl_hbm.at[pl.ds(c * (B * M * _NS) + hrow * _NS, hns)])

    return body(maskf, xyzf, featf, indsf)


def _mlp_body(x_ref, w0_ref, b0_ref, g0_ref, be0_ref, w1_ref, b1_ref, g1_ref,
              be1_ref, w2_ref, b2_ref, g2_ref, be2_ref, out_ref, grp_ref):
    sq_bn = jnp.sqrt(jnp.float32(1.0 + _BN_EPS))
    h = jnp.transpose(x_ref[...], (1, 0))
    grp_ref[...] = h.reshape(_RB // _NS, _NS, 4)
    for wr, br, gr, ber in ((w0_ref, b0_ref, g0_ref, be0_ref),
                            (w1_ref, b1_ref, g1_ref, be1_ref),
                            (w2_ref, b2_ref, g2_ref, be2_ref)):
        y = lax.dot_general(
            h.astype(jnp.bfloat16), wr[...].astype(jnp.bfloat16),
            (((1,), (0,)), ((), ())), preferred_element_type=jnp.float32)
        y = y + br[...]
        y = gr[...] * (y / sq_bn) + ber[...]
        h = jnp.clip(y, 0.0, 6.0)
    hm = h.reshape(_RB // _NS, _NS, h.shape[-1])
    out_ref[...] = jnp.max(hm, axis=1)


def _mlp_call(x, params):
    rows = x.shape[1]
    full = lambda shape: pl.BlockSpec(shape, lambda i: (0,) * len(shape))
    in_specs = [pl.BlockSpec((4, _RB), lambda i: (0, i))]
    args = [x]
    for (w, b, g, be) in params:
        in_specs += [full(w.shape), full(b.shape), full(g.shape), full(be.shape)]
        args += [w, b, g, be]
    return pl.pallas_call(
        _mlp_body,
        grid=(rows // _RB,),
        in_specs=in_specs,
        out_specs=[pl.BlockSpec((_RB // _NS, 128), lambda i: (i, 0)),
                   pl.BlockSpec((_RB // _NS, _NS, 4), lambda i: (i, 0, 0))],
        out_shape=[jax.ShapeDtypeStruct((rows // _NS, 128), jnp.float32),
                   jax.ShapeDtypeStruct((rows // _NS, _NS, 4), jnp.float32)],
    )(*args)


def _bit_weights():
    p = np.zeros((_NB, _NB // 16), np.float32)
    n = np.arange(_NB)
    p[n, n // 16] = 2.0 ** (n % 16)
    return jnp.asarray(p, dtype=jnp.bfloat16)


def kernel(xyz, features, inds, W0, b0, g0, be0, W1, b1, g1, be1, W2, b2, g2, be2):
    B, N, _ = xyz.shape
    M = inds.shape[1]
    xyzf = xyz.reshape(-1)
    featf = features.reshape(-1)
    indsf = inds.reshape(-1)

    nxyzp_flat, xyzt_flat = _centers_call(xyzf, indsf, B, N, M)
    xyz_t = xyzt_flat.reshape(B, 3, N)

    maskw, new_xyz = _mask_pallas(nxyzp_flat.reshape(B, 3, M), xyz_t,
                                  _bit_weights())

    xplanes = _group_call(maskw.reshape(-1), xyzf, featf, indsf, B, N, M)

    params = [(W0, b0.reshape(1, -1), g0.reshape(1, -1), be0.reshape(1, -1)),
              (W1, b1.reshape(1, -1), g1.reshape(1, -1), be1.reshape(1, -1)),
              (W2, b2.reshape(1, -1), g2.reshape(1, -1), be2.reshape(1, -1))]
    nf, grouped = _mlp_call(xplanes.reshape(4, B * M * _NS), params)
    new_features = nf.reshape(B, M, 128)
    grouped_features = grouped.reshape(B, M, _NS, 4)

    return (new_xyz, new_features, inds, grouped_features)


# speedup vs baseline: 1.4053x; 1.0737x over previous
import functools

import jax
import jax.numpy as jnp
import numpy as np
from jax import lax
from jax.experimental import pallas as pl
from jax.experimental.pallas import tpu as pltpu
from jax.experimental.pallas import tpu_sc as plsc

_RADIUS = 0.4
_R2 = _RADIUS * _RADIUS
_NS = 64
_BN_EPS = 1e-3

_NW = 32
_GRP = 8

_MB = 512
_NB = 2048
_RB = 8192


def _mesh():
    return plsc.VectorSubcoreMesh(core_axis_name="c", subcore_axis_name="s")


_SC_PARAMS = pltpu.CompilerParams(needs_layout_passes=False)


def _centers_call(xyzf, indsf, B, N, M):
    wpb = _NW // B
    mpw = M // wpb
    npw = N // wpb

    @functools.partial(
        pl.kernel,
        out_type=(jax.ShapeDtypeStruct((3 * B * M,), jnp.float32),
                  jax.ShapeDtypeStruct((B * 3 * N,), jnp.float32)),
        mesh=_mesh(),
        compiler_params=_SC_PARAMS,
        scratch_types=[
            pltpu.VMEM((3 * N,), jnp.float32),
            pltpu.VMEM((mpw,), jnp.int32),
            pltpu.VMEM((mpw * 3,), jnp.float32),
            pltpu.VMEM((npw,), jnp.float32),
        ],
    )
    def body(xyz_hbm, inds_hbm, nxyz_hbm, xyzt_hbm, pb, ib, nb, tb):
        w = lax.axis_index("s") * 2 + lax.axis_index("c")
        b = w // wpb
        k = w % wpb
        pltpu.sync_copy(xyz_hbm.at[pl.ds(b * 3 * N, 3 * N)], pb)
        pltpu.sync_copy(inds_hbm.at[pl.ds(b * M + k * mpw, mpw)], ib)
        iota = lax.iota(jnp.int32, 16)
        for j in range(mpw // 16):
            civ = ib[pl.ds(j * 16, 16)] * 3
            for c in range(3):
                nb[pl.ds(c * mpw + j * 16, 16)] = plsc.load_gather(pb, [civ + c])
        for c in range(3):
            pltpu.sync_copy(nb.at[pl.ds(c * mpw, mpw)],
                            nxyz_hbm.at[pl.ds((b * 3 + c) * M + k * mpw, mpw)])
        for c in range(3):
            for j in range(npw // 16):
                src = (k * npw + j * 16 + iota) * 3 + c
                tb[pl.ds(j * 16, 16)] = plsc.load_gather(pb, [src])
            pltpu.sync_copy(tb, xyzt_hbm.at[pl.ds((b * 3 + c) * N + k * npw, npw)])

    return body(xyzf, indsf)


def _sq_body(nx_ref, xt_ref, p_ref, out_ref, nxyz_ref):
    ap = nx_ref[0]
    a = jnp.transpose(ap, (1, 0))
    bt = xt_ref[0]
    nxyz_ref[0] = a
    a2 = a.astype(jnp.bfloat16) * jnp.bfloat16(2.0)
    dot2 = lax.dot_general(
        a2, bt.astype(jnp.bfloat16),
        (((1,), (0,)), ((), ())), preferred_element_type=jnp.float32)
    cn = jnp.sum(a * a, axis=1)[:, None]
    pn = jnp.sum(bt * bt, axis=0)[None, :]
    sq = cn + pn - dot2
    mb = (sq < _R2).astype(jnp.bfloat16)
    packed = lax.dot_general(
        mb, p_ref[...],
        (((1,), (0,)), ((), ())), preferred_element_type=jnp.float32)
    out_ref[0] = packed.astype(jnp.int32)


def _mask_pallas(nxyz_planes, xyz_t, pmat):
    B, _, M = nxyz_planes.shape
    N = xyz_t.shape[2]
    nw = N // 16
    return pl.pallas_call(
        _sq_body,
        grid=(B, M // _MB, N // _NB),
        in_specs=[
            pl.BlockSpec((1, 3, _MB), lambda b, i, j: (b, 0, i)),
            pl.BlockSpec((1, 3, _NB), lambda b, i, j: (b, 0, j)),
            pl.BlockSpec((_NB, _NB // 16), lambda b, i, j: (0, 0)),
        ],
        out_specs=[pl.BlockSpec((1, _MB, _NB // 16), lambda b, i, j: (b, i, j)),
                   pl.BlockSpec((1, _MB, 3), lambda b, i, j: (b, i, 0))],
        out_shape=[jax.ShapeDtypeStruct((B, M, nw), jnp.int32),
                   jax.ShapeDtypeStruct((B, M, 3), jnp.float32)],
    )(nxyz_planes, xyz_t, pmat)


def _group_call(maskf, xyzf, featf, indsf, B, N, M):
    wpb = _NW // B
    mpw = M // wpb
    nw = N // 16
    nwv = nw // 16

    @functools.partial(
        pl.kernel,
        out_type=jax.ShapeDtypeStruct((4 * B * M * _NS,), jnp.float32),
        mesh=_mesh(),
        compiler_params=_SC_PARAMS,
        scratch_types=[
            pltpu.VMEM((3 * N,), jnp.float32),
            pltpu.VMEM((N,), jnp.float32),
            pltpu.VMEM((mpw,), jnp.int32),
            pltpu.VMEM((mpw + 16,), jnp.float32),
            pltpu.VMEM((mpw + 16,), jnp.float32),
            pltpu.VMEM((mpw + 16,), jnp.float32),
            pltpu.VMEM((_GRP * nw,), jnp.int32),
            pltpu.VMEM((_NS + 16,), jnp.int32),
            pltpu.VMEM((2 * mpw * _NS,), jnp.float32),
        ],
    )
    def body(mask_hbm, xyz_hbm, feat_hbm, inds_hbm, xpl_hbm,
             pb, fs, ib, cxb, cyb, czb, mkb, idxb, px):
        w = lax.axis_index("s") * 2 + lax.axis_index("c")
        b = w // wpb
        k = w % wpb
        pltpu.sync_copy(xyz_hbm.at[pl.ds(b * 3 * N, 3 * N)], pb)
        pltpu.sync_copy(feat_hbm.at[pl.ds(b * N, N)], fs)
        pltpu.sync_copy(inds_hbm.at[pl.ds(b * M + k * mpw, mpw)], ib)
        iota = lax.iota(jnp.int32, 16)
        for j in range(mpw // 16):
            civ = ib[pl.ds(j * 16, 16)] * 3
            cxb[pl.ds(j * 16, 16)] = plsc.load_gather(pb, [civ])
            cyb[pl.ds(j * 16, 16)] = plsc.load_gather(pb, [civ + 1])
            czb[pl.ds(j * 16, 16)] = plsc.load_gather(pb, [civ + 2])
        row0 = b * M + k * mpw

        def make_group_body(h):
          def group_body(g, carry0):
            pltpu.sync_copy(
                mask_hbm.at[pl.ds((row0 + h * (mpw // 2) + g * _GRP) * nw,
                                  _GRP * nw)], mkb)

            def center_body(ci8, carry):
                ci = h * (mpw // 2) + g * _GRP + ci8
                cl = g * _GRP + ci8
                cx = cxb[pl.ds(ci, 16)][0]
                cy = cyb[pl.ds(ci, 16)][0]
                cz = czb[pl.ds(ci, 16)][0]

                def cond(st):
                    wv_i, cnt = st
                    return (cnt < _NS) & (wv_i < nwv)

                def wbody(st):
                    wv_i, cnt = st
                    wv = mkb[pl.ds(ci8 * nw + wv_i * 16, 16)]
                    base = wv_i * 256
                    for j in range(16):
                        m = ((wv[j] >> iota) & 1) == 1
                        pidx = base + j * 16 + iota
                        plsc.store_compressed(
                            idxb.at[pl.ds(jnp.minimum(cnt, _NS), 16)],
                            pidx, mask=m)
                        cnt = cnt + plsc.all_reduce_population_count(m)[0]
                    return wv_i + 1, cnt

                _, cnt = lax.while_loop(cond, wbody,
                                        (jnp.int32(0), jnp.int32(0)))
                first = idxb[pl.ds(0, 16)][0]
                for t in range(_NS // 16):
                    pos = iota + t * 16
                    iv = idxb[pl.ds(t * 16, 16)]
                    iv = jnp.where(pos < cnt, iv, first)
                    iv3 = iv * 3
                    gx = (plsc.load_gather(pb, [iv3]) - cx) / _RADIUS
                    gy = (plsc.load_gather(pb, [iv3 + 1]) - cy) / _RADIUS
                    gz = (plsc.load_gather(pb, [iv3 + 2]) - cz) / _RADIUS
                    gf = plsc.load_gather(fs, [iv])
                    hns = (mpw // 2) * _NS
                    rbase = cl * _NS + t * 16
                    px[pl.ds(0 * hns + rbase, 16)] = gx
                    px[pl.ds(1 * hns + rbase, 16)] = gy
                    px[pl.ds(2 * hns + rbase, 16)] = gz
                    px[pl.ds(3 * hns + rbase, 16)] = gf
                return carry

            return lax.fori_loop(0, _GRP, center_body, carry0)
          return group_body

        hns = (mpw // 2) * _NS
        for h in range(2):
            lax.fori_loop(0, mpw // 2 // _GRP, make_group_body(h), jnp.int32(0))
            hrow = row0 + h * (mpw // 2)
            for c in range(4):
                pltpu.sync_copy(
                    px.at[pl.ds(c * hns, hns)],
                    xpl_hbm.at[pl.ds(c * (B * M * _NS) + hrow * _NS, hns)])

    return body(maskf, xyzf, featf, indsf)


def _mlp_body(x_ref, w0_ref, b0_ref, g0_ref, be0_ref, w1_ref, b1_ref, g1_ref,
              be1_ref, w2_ref, b2_ref, g2_ref, be2_ref, out_ref, grp_ref):
    sq_bn = jnp.sqrt(jnp.float32(1.0 + _BN_EPS))
    h = jnp.transpose(x_ref[...], (1, 0))
    grp_ref[...] = h.reshape(_RB // _NS, _NS, 4)
    for wr, br, gr, ber in ((w0_ref, b0_ref, g0_ref, be0_ref),
                            (w1_ref, b1_ref, g1_ref, be1_ref),
                            (w2_ref, b2_ref, g2_ref, be2_ref)):
        y = lax.dot_general(
            h.astype(jnp.bfloat16), wr[...].astype(jnp.bfloat16),
            (((1,), (0,)), ((), ())), preferred_element_type=jnp.float32)
        y = y + br[...]
        y = gr[...] * (y / sq_bn) + ber[...]
        h = jnp.clip(y, 0.0, 6.0)
    hm = h.reshape(_RB // _NS, _NS, h.shape[-1])
    out_ref[...] = jnp.max(hm, axis=1)


def _mlp_call(x, params):
    rows = x.shape[1]
    full = lambda shape: pl.BlockSpec(shape, lambda i: (0,) * len(shape))
    in_specs = [pl.BlockSpec((4, _RB), lambda i: (0, i))]
    args = [x]
    for (w, b, g, be) in params:
        in_specs += [full(w.shape), full(b.shape), full(g.shape), full(be.shape)]
        args += [w, b, g, be]
    return pl.pallas_call(
        _mlp_body,
        grid=(rows // _RB,),
        in_specs=in_specs,
        out_specs=[pl.BlockSpec((_RB // _NS, 128), lambda i: (i, 0)),
                   pl.BlockSpec((_RB // _NS, _NS, 4), lambda i: (i, 0, 0))],
        out_shape=[jax.ShapeDtypeStruct((rows // _NS, 128), jnp.float32),
                   jax.ShapeDtypeStruct((rows // _NS, _NS, 4), jnp.float32)],
    )(*args)


def _bit_weights():
    p = np.zeros((_NB, _NB // 16), np.float32)
    n = np.arange(_NB)
    p[n, n // 16] = 2.0 ** (n % 16)
    return jnp.asarray(p, dtype=jnp.bfloat16)


def kernel(xyz, features, inds, W0, b0, g0, be0, W1, b1, g1, be1, W2, b2, g2, be2):
    B, N, _ = xyz.shape
    M = inds.shape[1]
    xyzf = xyz.reshape(-1)
    featf = features.reshape(-1)
    indsf = inds.reshape(-1)

    nxyzp_flat, xyzt_flat = _centers_call(xyzf, indsf, B, N, M)
    xyz_t = xyzt_flat.reshape(B, 3, N)

    maskw, new_xyz = _mask_pallas(nxyzp_flat.reshape(B, 3, M), xyz_t,
                                  _bit_weights())

    xplanes = _group_call(maskw.reshape(-1), xyzf, featf, indsf, B, N, M)

    params = [(W0, b0.reshape(1, -1), g0.reshape(1, -1), be0.reshape(1, -1)),
              (W1, b1.reshape(1, -1), g1.reshape(1, -1), be1.reshape(1, -1)),
              (W2, b2.reshape(1, -1), g2.reshape(1, -1), be2.reshape(1, -1))]
    nf, grouped = _mlp_call(xplanes.reshape(4, B * M * _NS), params)
    new_features = nf.reshape(B, M, 128)
    grouped_features = grouped.reshape(B, M, _NS, 4)

    return (new_xyz, new_features, inds, grouped_features)


# speedup vs baseline: 1.4522x; 1.0333x over previous
import functools

import jax
import jax.numpy as jnp
import numpy as np
from jax import lax
from jax.experimental import pallas as pl
from jax.experimental.pallas import tpu as pltpu
from jax.experimental.pallas import tpu_sc as plsc

_RADIUS = 0.4
_R2 = _RADIUS * _RADIUS
_NS = 64
_BN_EPS = 1e-3

_NW = 32
_GRP = 8

_MB = 1024
_NB = 2048
_RB = 16384


def _mesh():
    return plsc.VectorSubcoreMesh(core_axis_name="c", subcore_axis_name="s")


_SC_PARAMS = pltpu.CompilerParams(needs_layout_passes=False)


def _centers_call(xyzf, indsf, B, N, M):
    wpb = _NW // B
    mpw = M // wpb
    npw = N // wpb

    @functools.partial(
        pl.kernel,
        out_type=(jax.ShapeDtypeStruct((3 * B * M,), jnp.float32),
                  jax.ShapeDtypeStruct((B * 3 * N,), jnp.float32)),
        mesh=_mesh(),
        compiler_params=_SC_PARAMS,
        scratch_types=[
            pltpu.VMEM((3 * N,), jnp.float32),
            pltpu.VMEM((mpw,), jnp.int32),
            pltpu.VMEM((mpw * 3,), jnp.float32),
            pltpu.VMEM((npw,), jnp.float32),
        ],
    )
    def body(xyz_hbm, inds_hbm, nxyz_hbm, xyzt_hbm, pb, ib, nb, tb):
        w = lax.axis_index("s") * 2 + lax.axis_index("c")
        b = w // wpb
        k = w % wpb
        pltpu.sync_copy(xyz_hbm.at[pl.ds(b * 3 * N, 3 * N)], pb)
        pltpu.sync_copy(inds_hbm.at[pl.ds(b * M + k * mpw, mpw)], ib)
        iota = lax.iota(jnp.int32, 16)
        for j in range(mpw // 16):
            civ = ib[pl.ds(j * 16, 16)] * 3
            for c in range(3):
                nb[pl.ds(c * mpw + j * 16, 16)] = plsc.load_gather(pb, [civ + c])
        for c in range(3):
            pltpu.sync_copy(nb.at[pl.ds(c * mpw, mpw)],
                            nxyz_hbm.at[pl.ds((b * 3 + c) * M + k * mpw, mpw)])
        for c in range(3):
            for j in range(npw // 16):
                src = (k * npw + j * 16 + iota) * 3 + c
                tb[pl.ds(j * 16, 16)] = plsc.load_gather(pb, [src])
            pltpu.sync_copy(tb, xyzt_hbm.at[pl.ds((b * 3 + c) * N + k * npw, npw)])

    return body(xyzf, indsf)


def _sq_body(nx_ref, xt_ref, p_ref, out_ref, nxyz_ref):
    ap = nx_ref[0]
    a = jnp.transpose(ap, (1, 0))
    bt = xt_ref[0]
    nxyz_ref[0] = a
    a2 = a.astype(jnp.bfloat16) * jnp.bfloat16(2.0)
    dot2 = lax.dot_general(
        a2, bt.astype(jnp.bfloat16),
        (((1,), (0,)), ((), ())), preferred_element_type=jnp.float32)
    cn = jnp.sum(a * a, axis=1)[:, None]
    pn = jnp.sum(bt * bt, axis=0)[None, :]
    sq = cn + pn - dot2
    mb = (sq < _R2).astype(jnp.bfloat16)
    packed = lax.dot_general(
        mb, p_ref[...],
        (((1,), (0,)), ((), ())), preferred_element_type=jnp.float32)
    out_ref[0] = packed.astype(jnp.int32)


def _mask_pallas(nxyz_planes, xyz_t, pmat):
    B, _, M = nxyz_planes.shape
    N = xyz_t.shape[2]
    nw = N // 16
    return pl.pallas_call(
        _sq_body,
        grid=(B, M // _MB, N // _NB),
        in_specs=[
            pl.BlockSpec((1, 3, _MB), lambda b, i, j: (b, 0, i)),
            pl.BlockSpec((1, 3, _NB), lambda b, i, j: (b, 0, j)),
            pl.BlockSpec((_NB, _NB // 16), lambda b, i, j: (0, 0)),
        ],
        out_specs=[pl.BlockSpec((1, _MB, _NB // 16), lambda b, i, j: (b, i, j)),
                   pl.BlockSpec((1, _MB, 3), lambda b, i, j: (b, i, 0))],
        out_shape=[jax.ShapeDtypeStruct((B, M, nw), jnp.int32),
                   jax.ShapeDtypeStruct((B, M, 3), jnp.float32)],
    )(nxyz_planes, xyz_t, pmat)


def _group_call(maskf, xyzf, featf, indsf, B, N, M):
    wpb = _NW // B
    mpw = M // wpb
    nw = N // 16
    nwv = nw // 16

    @functools.partial(
        pl.kernel,
        out_type=jax.ShapeDtypeStruct((4 * B * M * _NS,), jnp.float32),
        mesh=_mesh(),
        compiler_params=_SC_PARAMS,
        scratch_types=[
            pltpu.VMEM((3 * N,), jnp.float32),
            pltpu.VMEM((N,), jnp.float32),
            pltpu.VMEM((mpw,), jnp.int32),
            pltpu.VMEM((mpw + 16,), jnp.float32),
            pltpu.VMEM((mpw + 16,), jnp.float32),
            pltpu.VMEM((mpw + 16,), jnp.float32),
            pltpu.VMEM((_GRP * nw,), jnp.int32),
            pltpu.VMEM((_NS + 16,), jnp.int32),
            pltpu.VMEM((2 * mpw * _NS,), jnp.float32),
        ],
    )
    def body(mask_hbm, xyz_hbm, feat_hbm, inds_hbm, xpl_hbm,
             pb, fs, ib, cxb, cyb, czb, mkb, idxb, px):
        w = lax.axis_index("s") * 2 + lax.axis_index("c")
        b = w // wpb
        k = w % wpb
        pltpu.sync_copy(xyz_hbm.at[pl.ds(b * 3 * N, 3 * N)], pb)
        pltpu.sync_copy(feat_hbm.at[pl.ds(b * N, N)], fs)
        pltpu.sync_copy(inds_hbm.at[pl.ds(b * M + k * mpw, mpw)], ib)
        iota = lax.iota(jnp.int32, 16)
        for j in range(mpw // 16):
            civ = ib[pl.ds(j * 16, 16)] * 3
            cxb[pl.ds(j * 16, 16)] = plsc.load_gather(pb, [civ])
            cyb[pl.ds(j * 16, 16)] = plsc.load_gather(pb, [civ + 1])
            czb[pl.ds(j * 16, 16)] = plsc.load_gather(pb, [civ + 2])
        row0 = b * M + k * mpw

        def make_group_body(h):
          def group_body(g, carry0):
            pltpu.sync_copy(
                mask_hbm.at[pl.ds((row0 + h * (mpw // 2) + g * _GRP) * nw,
                                  _GRP * nw)], mkb)

            def center_body(ci8, carry):
                ci = h * (mpw // 2) + g * _GRP + ci8
                cl = g * _GRP + ci8
                cx = cxb[pl.ds(ci, 16)][0]
                cy = cyb[pl.ds(ci, 16)][0]
                cz = czb[pl.ds(ci, 16)][0]

                def cond(st):
                    wv_i, cnt = st
                    return (cnt < _NS) & (wv_i < nwv)

                def wbody(st):
                    wv_i, cnt = st
                    wv = mkb[pl.ds(ci8 * nw + wv_i * 16, 16)]
                    base = wv_i * 256
                    for j in range(16):
                        m = ((wv[j] >> iota) & 1) == 1
                        pidx = base + j * 16 + iota
                        plsc.store_compressed(
                            idxb.at[pl.ds(jnp.minimum(cnt, _NS), 16)],
                            pidx, mask=m)
                        cnt = cnt + plsc.all_reduce_population_count(m)[0]
                    return wv_i + 1, cnt

                _, cnt = lax.while_loop(cond, wbody,
                                        (jnp.int32(0), jnp.int32(0)))
                first = idxb[pl.ds(0, 16)][0]
                for t in range(_NS // 16):
                    pos = iota + t * 16
                    iv = idxb[pl.ds(t * 16, 16)]
                    iv = jnp.where(pos < cnt, iv, first)
                    iv3 = iv * 3
                    gx = (plsc.load_gather(pb, [iv3]) - cx) / _RADIUS
                    gy = (plsc.load_gather(pb, [iv3 + 1]) - cy) / _RADIUS
                    gz = (plsc.load_gather(pb, [iv3 + 2]) - cz) / _RADIUS
                    gf = plsc.load_gather(fs, [iv])
                    hns = (mpw // 2) * _NS
                    rbase = cl * _NS + t * 16
                    px[pl.ds(0 * hns + rbase, 16)] = gx
                    px[pl.ds(1 * hns + rbase, 16)] = gy
                    px[pl.ds(2 * hns + rbase, 16)] = gz
                    px[pl.ds(3 * hns + rbase, 16)] = gf
                return carry

            return lax.fori_loop(0, _GRP, center_body, carry0)
          return group_body

        hns = (mpw // 2) * _NS
        for h in range(2):
            lax.fori_loop(0, mpw // 2 // _GRP, make_group_body(h), jnp.int32(0))
            hrow = row0 + h * (mpw // 2)
            for c in range(4):
                pltpu.sync_copy(
                    px.at[pl.ds(c * hns, hns)],
                    xpl_hbm.at[pl.ds(c * (B * M * _NS) + hrow * _NS, hns)])

    return body(maskf, xyzf, featf, indsf)


def _mlp_body(x_ref, w0_ref, b0_ref, g0_ref, be0_ref, w1_ref, b1_ref, g1_ref,
              be1_ref, w2_ref, b2_ref, g2_ref, be2_ref, out_ref, grp_ref):
    sq_bn = jnp.sqrt(jnp.float32(1.0 + _BN_EPS))
    h = jnp.transpose(x_ref[...], (1, 0))
    grp_ref[...] = h.reshape(_RB // _NS, _NS, 4)
    for wr, br, gr, ber in ((w0_ref, b0_ref, g0_ref, be0_ref),
                            (w1_ref, b1_ref, g1_ref, be1_ref),
                            (w2_ref, b2_ref, g2_ref, be2_ref)):
        y = lax.dot_general(
            h.astype(jnp.bfloat16), wr[...].astype(jnp.bfloat16),
            (((1,), (0,)), ((), ())), preferred_element_type=jnp.float32)
        y = y + br[...]
        y = gr[...] * (y / sq_bn) + ber[...]
        h = jnp.clip(y, 0.0, 6.0)
    hm = h.reshape(_RB // _NS, _NS, h.shape[-1])
    out_ref[...] = jnp.max(hm, axis=1)


def _mlp_call(x, params):
    rows = x.shape[1]
    full = lambda shape: pl.BlockSpec(shape, lambda i: (0,) * len(shape))
    in_specs = [pl.BlockSpec((4, _RB), lambda i: (0, i))]
    args = [x]
    for (w, b, g, be) in params:
        in_specs += [full(w.shape), full(b.shape), full(g.shape), full(be.shape)]
        args += [w, b, g, be]
    return pl.pallas_call(
        _mlp_body,
        grid=(rows // _RB,),
        in_specs=in_specs,
        out_specs=[pl.BlockSpec((_RB // _NS, 128), lambda i: (i, 0)),
                   pl.BlockSpec((_RB // _NS, _NS, 4), lambda i: (i, 0, 0))],
        out_shape=[jax.ShapeDtypeStruct((rows // _NS, 128), jnp.float32),
                   jax.ShapeDtypeStruct((rows // _NS, _NS, 4), jnp.float32)],
    )(*args)


def _bit_weights():
    p = np.zeros((_NB, _NB // 16), np.float32)
    n = np.arange(_NB)
    p[n, n // 16] = 2.0 ** (n % 16)
    return jnp.asarray(p, dtype=jnp.bfloat16)


def kernel(xyz, features, inds, W0, b0, g0, be0, W1, b1, g1, be1, W2, b2, g2, be2):
    B, N, _ = xyz.shape
    M = inds.shape[1]
    xyzf = xyz.reshape(-1)
    featf = features.reshape(-1)
    indsf = inds.reshape(-1)

    nxyzp_flat, xyzt_flat = _centers_call(xyzf, indsf, B, N, M)
    xyz_t = xyzt_flat.reshape(B, 3, N)

    maskw, new_xyz = _mask_pallas(nxyzp_flat.reshape(B, 3, M), xyz_t,
                                  _bit_weights())

    xplanes = _group_call(maskw.reshape(-1), xyzf, featf, indsf, B, N, M)

    params = [(W0, b0.reshape(1, -1), g0.reshape(1, -1), be0.reshape(1, -1)),
              (W1, b1.reshape(1, -1), g1.reshape(1, -1), be1.reshape(1, -1)),
              (W2, b2.reshape(1, -1), g2.reshape(1, -1), be2.reshape(1, -1))]
    nf, grouped = _mlp_call(xplanes.reshape(4, B * M * _NS), params)
    new_features = nf.reshape(B, M, 128)
    grouped_features = grouped.reshape(B, M, _NS, 4)

    return (new_xyz, new_features, inds, grouped_features)
